# Initial kernel scaffold; baseline (speedup 1.0000x reference)
import jax
import jax.numpy as jnp
from jax.experimental import pallas as pl


def kernel(es, edge_index, seeds, n_iter, W_h, W_m, b):
    raise NotImplementedError("write your pallas kernel here")


# speedup vs baseline: 13.9433x; 13.9433x over previous
import functools

import jax
import jax.numpy as jnp
from jax import lax
from jax.experimental import pallas as pl
from jax.experimental.pallas import tpu as pltpu
from jax.experimental.pallas import tpu_sc as plsc

N = 10000
E = 320000
D = 128
C = 8
K = 32
MIN_MATCH = 2
ITERS = 3

NW = 32
EPW = E // NW
GROUPS = EPW // 16

def _sc_counts_body(edge_hbm, bits_hbm, out_hbm, src_v, dst_v, bits_v, acc_v):
    wid = lax.axis_index("s") * 2 + lax.axis_index("c")
    base = wid * EPW
    pltpu.sync_copy(edge_hbm.at[pl.ds(base, EPW)], src_v)
    pltpu.sync_copy(edge_hbm.at[pl.ds(E + base, EPW)], dst_v)
    pltpu.sync_copy(bits_hbm, bits_v)

    zeros16 = jnp.zeros((16,), jnp.float32)

    def zero_body(i, carry):
        acc_v[pl.ds(i * 16, 16)] = zeros16
        return carry

    lax.fori_loop(0, C * N // 16, zero_body, 0)

    ones16 = jnp.ones((16,), jnp.float32)

    def edge_body(g, carry):
        s = src_v[pl.ds(g * 16, 16)]
        d = dst_v[pl.ds(g * 16, 16)]
        bv = plsc.load_gather(bits_v, [s])

        @pl.when(jnp.any(bv != 0))
        def _scatter():
            for c in range(C):
                on = (bv >> c) & 1
                plsc.addupdate_scatter(acc_v, [d + (c * N)], ones16,
                                       mask=(on != 0))

        return carry

    lax.fori_loop(0, GROUPS, edge_body, 0)
    pltpu.sync_copy(acc_v, out_hbm.at[wid])


@functools.cache
def _sc_counts():
    mesh = plsc.VectorSubcoreMesh(core_axis_name="c", subcore_axis_name="s",
                                  num_cores=2, num_subcores=16)
    return pl.kernel(
        _sc_counts_body,
        out_type=jax.ShapeDtypeStruct((NW, C * N), jnp.float32),
        mesh=mesh,
        compiler_params=pltpu.CompilerParams(needs_layout_passes=False),
        scratch_types=[
            pltpu.VMEM((EPW,), jnp.int32),
            pltpu.VMEM((EPW,), jnp.int32),
            pltpu.VMEM((N,), jnp.int32),
            pltpu.VMEM((C * N,), jnp.float32),
        ],
    )


def _reduce_body(p_ref, o_ref):
    o_ref[...] = jnp.sum(p_ref[...], axis=0)


def _prep_body(es_ref, o_ref):
    x = es_ref[...]
    nrm = jnp.sqrt(jnp.sum(x * x, axis=1, keepdims=True))
    o_ref[...] = x / (nrm + 1e-8)


def _step_body(es_ref, esnt_ref, counts_ref, bits_ref, wh_ref, wm_ref, b_ref,
               hx_ref, lidx_ref, lmask_ref,
               probs_o, tidx_o, hx_o, sel_o):
    rows = []
    for c in range(C):
        acc = jnp.zeros((1, D), jnp.float32)
        den = jnp.float32(0.0)
        for j in range(K):
            idx = lidx_ref[c, j]
            w = lmask_ref[c, j]
            acc = acc + es_ref[pl.ds(idx, 1), :] * w
            den = den + w
        rows.append(acc / jnp.maximum(den, 1.0))
    m = jnp.concatenate(rows, axis=0)
    hx = jnp.tanh(
        jnp.dot(hx_ref[...], wh_ref[...], preferred_element_type=jnp.float32)
        + jnp.dot(m, wm_ref[...], preferred_element_type=jnp.float32)
        + b_ref[...])
    hx_o[...] = hx

    hxn = hx / (jnp.sqrt(jnp.sum(hx * hx, axis=1, keepdims=True)) + 1e-8)
    scores = jnp.dot(hxn, esnt_ref[...],
                     preferred_element_type=jnp.float32)

    known = bits_ref[...] != 0
    cate_valid = (counts_ref[...] >= jnp.float32(MIN_MATCH)) & (~known)
    vals = jnp.where(cate_valid, scores, jnp.float32(-1e9))

    iota = lax.broadcasted_iota(jnp.int32, (C, N), 1)
    tvals, tidxs = [], []
    for _ in range(K):
        mx = jnp.max(vals, axis=1, keepdims=True)
        ix = jnp.min(jnp.where(vals == mx, iota, N), axis=1, keepdims=True)
        tvals.append(mx)
        tidxs.append(ix)
        vals = jnp.where(iota == ix, -jnp.inf, vals)
    tv = jnp.concatenate(tvals, axis=1)
    ti = jnp.concatenate(tidxs, axis=1)
    tidx_o[...] = ti
    probs_o[...] = jax.nn.sigmoid(tv)
    sel_o[...] = (tv > jnp.float32(-1e8)).astype(jnp.float32)


def _bits_body(bits_ref, idx_ref, sel_ref, o_ref):
    o_ref[...] = bits_ref[...]
    for c in range(C):
        for k in range(K):
            @pl.when(sel_ref[c, k] != 0.0)
            def _upd(c=c, k=k):
                idx = idx_ref[c, k]
                o_ref[pl.ds(idx, 1), :] = o_ref[pl.ds(idx, 1), :] | (1 << c)


_VMEM = functools.partial(pl.BlockSpec, memory_space=pltpu.VMEM)
_SMEM = functools.partial(pl.BlockSpec, memory_space=pltpu.SMEM)

_tc_reduce = pl.pallas_call(
    _reduce_body,
    in_specs=[_VMEM()],
    out_specs=_VMEM(),
    out_shape=jax.ShapeDtypeStruct((C, N), jnp.float32),
)

_tc_prep = pl.pallas_call(
    _prep_body,
    grid=(10,),
    in_specs=[pl.BlockSpec((N // 10, D), lambda i: (i, 0))],
    out_specs=pl.BlockSpec((N // 10, D), lambda i: (i, 0)),
    out_shape=jax.ShapeDtypeStruct((N, D), jnp.float32),
)

_tc_step = pl.pallas_call(
    _step_body,
    in_specs=[_VMEM(), _VMEM(), _VMEM(), _VMEM(), _VMEM(), _VMEM(), _VMEM(),
              _VMEM(), _SMEM(), _SMEM()],
    out_specs=[_VMEM(), _VMEM(), _VMEM(), _VMEM()],
    out_shape=[
        jax.ShapeDtypeStruct((C, K), jnp.float32),
        jax.ShapeDtypeStruct((C, K), jnp.int32),
        jax.ShapeDtypeStruct((C, D), jnp.float32),
        jax.ShapeDtypeStruct((C, K), jnp.float32),
    ],
)

_tc_bits = pl.pallas_call(
    _bits_body,
    in_specs=[_VMEM(), _SMEM(), _SMEM()],
    out_specs=_VMEM(),
    out_shape=jax.ShapeDtypeStruct((N, 1), jnp.int32),
)


def kernel(es, edge_index, seeds, n_iter, W_h, W_m, b):
    del n_iter
    es = es.astype(jnp.float32)
    b2 = b.reshape(1, D).astype(jnp.float32)

    esnt = _tc_prep(es).T
    bits_col = _tc_bits(jnp.zeros((N, 1), jnp.int32), seeds,
                        jnp.ones((C, K), jnp.float32))
    hx = jnp.zeros((C, D), jnp.float32)
    lidx = seeds
    lmask = jnp.ones((C, K), jnp.float32)

    probs_l, idx_l, hx_l = [], [], []
    for i in range(ITERS):
        partials = _sc_counts()(edge_index.reshape(2 * E), bits_col.reshape(N))
        counts = _tc_reduce(partials.reshape(NW, C, N))
        probs, tidx, hx, sel = _tc_step(es, esnt, counts, bits_col.reshape(1, N),
                                        W_h, W_m, b2, hx, lidx, lmask)
        if i < ITERS - 1:
            bits_col = _tc_bits(bits_col, tidx, sel)
        lidx, lmask = tidx, sel
        probs_l.append(probs)
        idx_l.append(tidx)
        hx_l.append(hx)

    return jnp.stack(probs_l), jnp.stack(idx_l), jnp.stack(hx_l)


# speedup vs baseline: 24.8498x; 1.7822x over previous
import functools

import jax
import jax.numpy as jnp
from jax import lax
from jax.experimental import pallas as pl
from jax.experimental.pallas import tpu as pltpu
from jax.experimental.pallas import tpu_sc as plsc

N = 10000
E = 320000
D = 128
C = 8
K = 32
MIN_MATCH = 2
ITERS = 3

NW = 32
EPW = E // NW
GROUPS = EPW // 16

def _sc_counts_body(edge_hbm, bits_hbm, out_hbm, src_v, dst_v, bits_v, acc_v,
                    sem):
    wid = lax.axis_index("s") * 2 + lax.axis_index("c")
    base = wid * EPW
    cp_s = pltpu.async_copy(edge_hbm.at[pl.ds(base, EPW)], src_v, sem)
    cp_d = pltpu.async_copy(edge_hbm.at[pl.ds(E + base, EPW)], dst_v, sem)
    cp_b = pltpu.async_copy(bits_hbm, bits_v, sem)

    zeros16 = jnp.zeros((16,), jnp.float32)

    for c in range(C):
        @plsc.parallel_loop(0, N // 16, 1, unroll=8)
        def _zero(j, c=c):
            acc_v[c, pl.ds(j * 16, 16)] = zeros16

    cp_s.wait()
    cp_d.wait()
    cp_b.wait()

    ones16 = jnp.ones((16,), jnp.float32)

    @plsc.parallel_loop(0, GROUPS, 1, unroll=4)
    def _edges(g):
        s = src_v[pl.ds(g * 16, 16)]
        d = dst_v[pl.ds(g * 16, 16)]
        bv = plsc.load_gather(bits_v, [s])
        for c in range(C):
            on = (bv >> c) & 1
            plsc.addupdate_scatter(acc_v, [jnp.full((16,), c, jnp.int32), d],
                                   ones16, mask=(on != 0))

    pltpu.sync_copy(acc_v, out_hbm.at[wid])


@functools.cache
def _sc_counts():
    mesh = plsc.VectorSubcoreMesh(core_axis_name="c", subcore_axis_name="s",
                                  num_cores=2, num_subcores=16)
    return pl.kernel(
        _sc_counts_body,
        out_type=jax.ShapeDtypeStruct((NW, C, N), jnp.float32),
        mesh=mesh,
        compiler_params=pltpu.CompilerParams(needs_layout_passes=False),
        scratch_types=[
            pltpu.VMEM((EPW,), jnp.int32),
            pltpu.VMEM((EPW,), jnp.int32),
            pltpu.VMEM((N,), jnp.int32),
            pltpu.VMEM((C, N), jnp.float32),
            pltpu.SemaphoreType.DMA,
        ],
    )


def _reduce_body(p_ref, o_ref):
    o_ref[...] = jnp.sum(p_ref[...], axis=0)


def _prep_body(es_ref, o_ref):
    x = es_ref[...]
    nrm = jnp.sqrt(jnp.sum(x * x, axis=1, keepdims=True))
    o_ref[...] = x / (nrm + 1e-8)


def _step_body(es_ref, esnt_ref, counts_ref, bits_ref, wh_ref, wm_ref, b_ref,
               hx_ref, lidx_ref, lmask_ref,
               probs_o, tidx_o, hx_o, sel_o):
    rows = []
    for c in range(C):
        acc = jnp.zeros((1, D), jnp.float32)
        den = jnp.float32(0.0)
        for j in range(K):
            idx = lidx_ref[c, j]
            w = lmask_ref[c, j]
            acc = acc + es_ref[pl.ds(idx, 1), :] * w
            den = den + w
        rows.append(acc / jnp.maximum(den, 1.0))
    m = jnp.concatenate(rows, axis=0)
    hx = jnp.tanh(
        jnp.dot(hx_ref[...], wh_ref[...], preferred_element_type=jnp.float32)
        + jnp.dot(m, wm_ref[...], preferred_element_type=jnp.float32)
        + b_ref[...])
    hx_o[...] = hx

    hxn = hx / (jnp.sqrt(jnp.sum(hx * hx, axis=1, keepdims=True)) + 1e-8)
    scores = jnp.dot(hxn, esnt_ref[...],
                     preferred_element_type=jnp.float32)

    known = bits_ref[...] != 0
    cate_valid = (counts_ref[...] >= jnp.float32(MIN_MATCH)) & (~known)
    vals = jnp.where(cate_valid, scores, jnp.float32(-1e9))

    iota = lax.broadcasted_iota(jnp.int32, (C, N), 1)
    tvals, tidxs = [], []
    for _ in range(K):
        mx = jnp.max(vals, axis=1, keepdims=True)
        ix = jnp.min(jnp.where(vals == mx, iota, N), axis=1, keepdims=True)
        tvals.append(mx)
        tidxs.append(ix)
        vals = jnp.where(iota == ix, -jnp.inf, vals)
    tv = jnp.concatenate(tvals, axis=1)
    ti = jnp.concatenate(tidxs, axis=1)
    tidx_o[...] = ti
    probs_o[...] = jax.nn.sigmoid(tv)
    sel_o[...] = (tv > jnp.float32(-1e8)).astype(jnp.float32)


def _bits_body(bits_ref, idx_ref, sel_ref, o_ref):
    o_ref[...] = bits_ref[...]
    for c in range(C):
        for k in range(K):
            @pl.when(sel_ref[c, k] != 0.0)
            def _upd(c=c, k=k):
                idx = idx_ref[c, k]
                o_ref[pl.ds(idx, 1), :] = o_ref[pl.ds(idx, 1), :] | (1 << c)


_VMEM = functools.partial(pl.BlockSpec, memory_space=pltpu.VMEM)
_SMEM = functools.partial(pl.BlockSpec, memory_space=pltpu.SMEM)

_tc_reduce = pl.pallas_call(
    _reduce_body,
    in_specs=[_VMEM()],
    out_specs=_VMEM(),
    out_shape=jax.ShapeDtypeStruct((C, N), jnp.float32),
)

_tc_prep = pl.pallas_call(
    _prep_body,
    grid=(10,),
    in_specs=[pl.BlockSpec((N // 10, D), lambda i: (i, 0))],
    out_specs=pl.BlockSpec((N // 10, D), lambda i: (i, 0)),
    out_shape=jax.ShapeDtypeStruct((N, D), jnp.float32),
)

_tc_step = pl.pallas_call(
    _step_body,
    in_specs=[_VMEM(), _VMEM(), _VMEM(), _VMEM(), _VMEM(), _VMEM(), _VMEM(),
              _VMEM(), _SMEM(), _SMEM()],
    out_specs=[_VMEM(), _VMEM(), _VMEM(), _VMEM()],
    out_shape=[
        jax.ShapeDtypeStruct((C, K), jnp.float32),
        jax.ShapeDtypeStruct((C, K), jnp.int32),
        jax.ShapeDtypeStruct((C, D), jnp.float32),
        jax.ShapeDtypeStruct((C, K), jnp.float32),
    ],
)

_tc_bits = pl.pallas_call(
    _bits_body,
    in_specs=[_VMEM(), _SMEM(), _SMEM()],
    out_specs=_VMEM(),
    out_shape=jax.ShapeDtypeStruct((N, 1), jnp.int32),
)


def kernel(es, edge_index, seeds, n_iter, W_h, W_m, b):
    del n_iter
    es = es.astype(jnp.float32)
    b2 = b.reshape(1, D).astype(jnp.float32)

    esnt = _tc_prep(es).T
    bits_col = _tc_bits(jnp.zeros((N, 1), jnp.int32), seeds,
                        jnp.ones((C, K), jnp.float32))
    hx = jnp.zeros((C, D), jnp.float32)
    lidx = seeds
    lmask = jnp.ones((C, K), jnp.float32)

    probs_l, idx_l, hx_l = [], [], []
    for i in range(ITERS):
        partials = _sc_counts()(edge_index.reshape(2 * E), bits_col.reshape(N))
        counts = _tc_reduce(partials)
        probs, tidx, hx, sel = _tc_step(es, esnt, counts, bits_col.reshape(1, N),
                                        W_h, W_m, b2, hx, lidx, lmask)
        if i < ITERS - 1:
            bits_col = _tc_bits(bits_col, tidx, sel)
        lidx, lmask = tidx, sel
        probs_l.append(probs)
        idx_l.append(tidx)
        hx_l.append(hx)

    return jnp.stack(probs_l), jnp.stack(idx_l), jnp.stack(hx_l)
